# baseline (speedup 1.0000x reference)
import functools

import jax
import jax.numpy as jnp
import numpy as np
from jax import lax
from jax.experimental import pallas as pl
from jax.experimental.pallas import tpu as pltpu
from jax.experimental.pallas import tpu_sc as plsc

B = 65536
T = 524288
NL = [16, 22, 30, 42, 58, 80, 111, 153, 212, 294, 406, 561, 776, 1072, 1482, 2048]
NLEV = len(NL)
K2 = np.int32(np.int64(2654435761) - (1 << 32))
K3 = np.int32(805459861)
MASK19 = np.int32(T - 1)

NC, NS, LANES = 2, 16, 16
NW = NC * NS
PTS = B // NW
CP = 128
GP = 128
NCHUNK = PTS // GP
GROUPS = GP // LANES
NIDX = NLEV * 8

f32 = jnp.float32
i32 = jnp.int32


def _tc_pack(tables):
    NB = 2048
    zero = np.int32(0)

    def body(in_r, out_r):
        blk = in_r[...]
        a = blk[0, :, 0, :]
        b = blk[0, :, 1, :]
        a32 = lax.bitcast_convert_type(a.astype(jnp.bfloat16), jnp.uint16).astype(jnp.uint32)
        b32 = lax.bitcast_convert_type(b.astype(jnp.bfloat16), jnp.uint16).astype(jnp.uint32)
        word = (b32 << jnp.uint32(16)) | a32
        out_r[...] = lax.bitcast_convert_type(word, i32)[None]

    return pl.pallas_call(
        body,
        grid=(NLEV, (T // 128) // NB),
        in_specs=[pl.BlockSpec((1, NB, 2, 128), lambda i, j: (i, j, zero, zero))],
        out_specs=pl.BlockSpec((1, NB, 128), lambda i, j: (i, j, zero)),
        out_shape=jax.ShapeDtypeStruct((NLEV, T // 128, 128), i32),
        compiler_params=pltpu.CompilerParams(
            dimension_semantics=("arbitrary", "arbitrary")),
    )(tables)


def _sc_features(xT, tpack):
    mesh = plsc.VectorSubcoreMesh(core_axis_name="c", subcore_axis_name="s")

    @functools.partial(
        pl.kernel,
        mesh=mesh,
        out_type=jax.ShapeDtypeStruct((B * 2 * NLEV,), f32),
        scratch_types=[
            pltpu.VMEM((3, PTS), f32),
            pltpu.VMEM((NIDX * GP,), i32),
            pltpu.VMEM((NIDX * GP,), i32),
            pltpu.VMEM((NIDX * GP,), i32),
            pltpu.VMEM((NIDX * GP,), i32),
            pltpu.VMEM((CP * 2 * NLEV,), f32),
            pltpu.SemaphoreType.DMA,
            pltpu.SemaphoreType.DMA,
        ],
    )
    def k(xT_hbm, tab_hbm, out_hbm, xyz_v, idxA, idxB, rowsA, rowsB, feat_v, semA, semB):
        wid = lax.axis_index("s") * NC + lax.axis_index("c")
        base = wid * PTS
        pltpu.sync_copy(xT_hbm.at[:, pl.ds(base, PTS)], xyz_v)

        def pass1(c, idx_v):
            cbase = c * GP

            def grp(g, carry1):
                p0 = g * LANES
                xs = xyz_v[0, pl.ds(cbase + p0, LANES)]
                ys = xyz_v[1, pl.ds(cbase + p0, LANES)]
                zs = xyz_v[2, pl.ds(cbase + p0, LANES)]
                for l in range(NLEV):
                    n = f32(NL[l])
                    ix = (xs * n).astype(i32)
                    iy = (ys * n).astype(i32)
                    iz = (zs * n).astype(i32)
                    hx0 = ix
                    hx1 = ix + 1
                    hy0 = iy * K2
                    hy1 = hy0 + K2
                    hz0 = iz * K3
                    hz1 = hz0 + K3
                    lbase = np.int32(l * T)
                    for v in range(8):
                        hx = hx1 if (v & 1) else hx0
                        hy = hy1 if (v >> 1) & 1 else hy0
                        hz = hz1 if (v >> 2) & 1 else hz0
                        e0 = ((hx ^ hy ^ hz) & MASK19) + lbase
                        s0 = (l * 8 + v) * GP
                        idx_v[pl.ds(s0 + p0, LANES)] = e0
                return carry1

            lax.fori_loop(i32(0), i32(GROUPS), grp, i32(0))

        def fire(idx_v, rows_v, sem):
            pltpu.async_copy(tab_hbm.at[idx_v], rows_v, sem)

        def wait(idx_v, rows_v, sem):
            pltpu.make_async_copy(tab_hbm.at[idx_v], rows_v, sem).wait()

        def pass2(c, rows_v, half):
            cbase = c * GP

            def grp(g, carry2):
                p0 = g * LANES
                xs = xyz_v[0, pl.ds(cbase + p0, LANES)]
                ys = xyz_v[1, pl.ds(cbase + p0, LANES)]
                zs = xyz_v[2, pl.ds(cbase + p0, LANES)]
                for l in range(NLEV):
                    n = f32(NL[l])
                    xn = xs * n
                    yn = ys * n
                    zn = zs * n
                    fx = xn - xn.astype(i32).astype(f32)
                    fy = yn - yn.astype(i32).astype(f32)
                    fz = zn - zn.astype(i32).astype(f32)
                    gx = f32(1.0) - fx
                    gy = f32(1.0) - fy
                    gz = f32(1.0) - fz
                    wxy = (gx * gy, fx * gy, gx * fy, fx * fy)
                    acc0 = jnp.zeros((LANES,), f32)
                    acc1 = jnp.zeros((LANES,), f32)
                    for v in range(8):
                        w = wxy[v & 3] * (fz if (v >> 2) & 1 else gz)
                        s0 = (l * 8 + v) * GP
                        r = rows_v[pl.ds(s0 + p0, LANES)]
                        f0 = lax.bitcast_convert_type(r << np.int32(16), f32)
                        f1 = lax.bitcast_convert_type(r & np.int32(-65536), f32)
                        acc0 = acc0 + f0 * w
                        acc1 = acc1 + f1 * w
                    feat_v[pl.ds((2 * l) * CP + p0, LANES)] = acc0
                    feat_v[pl.ds((2 * l + 1) * CP + p0, LANES)] = acc1
                return carry2

            lax.fori_loop(i32(0), i32(GROUPS), grp, i32(0))

        def flush(c):
            off = (wid * (PTS // CP) + c) * (CP * 2 * NLEV)
            pltpu.sync_copy(feat_v, out_hbm.at[pl.ds(off, CP * 2 * NLEV)])

        pass1(i32(0), idxA)
        fire(idxA, rowsA, semA)
        pass1(i32(1), idxB)

        def pair(i, carry):
            c = i * 2
            wait(idxA, rowsA, semA)
            fire(idxB, rowsB, semB)
            pass2(c, rowsA, 0)
            flush(c)
            pass1(c + 2, idxA)
            wait(idxB, rowsB, semB)
            fire(idxA, rowsA, semA)
            pass2(c + 1, rowsB, 0)
            flush(c + 1)
            pass1(c + 3, idxB)
            return carry

        lax.fori_loop(i32(0), i32(NCHUNK // 2 - 1), pair, i32(0))
        wait(idxA, rowsA, semA)
        fire(idxB, rowsB, semB)
        pass2(i32(NCHUNK - 2), rowsA, 0)
        flush(i32(NCHUNK - 2))
        wait(idxB, rowsB, semB)
        pass2(i32(NCHUNK - 1), rowsB, 0)
        flush(i32(NCHUNK - 1))

    return k(xT, tpack)


def _tc_heads(raw, xT3, dT3, w):
    TB = 4096
    C = TB // CP
    grid = (B // TB,)

    def dg1(wm, a):
        return lax.dot_general(wm, a, (((0,), (1,)), ((), ())),
                               preferred_element_type=f32)

    def dg0(wm, a):
        return lax.dot_general(wm, a, (((0,), (0,)), ((), ())),
                               preferred_element_type=f32)

    def body(raw_r, x_r, d_r,
             Wd1, bd1, Wd2, bd2, Wc1h, Wc1x, bc1, Wc2, bc2, Wc3, bc3,
             Ws1, bs1, Ws2, bs2, Ws3, bs3,
             color_r, sigma_r, seg_r):
        f = raw_r[...]
        xb = x_r[...]
        db = d_r[...]
        mask = ((jnp.abs(xb[0:1]) < f32(1.0))
                & (jnp.abs(xb[1:2]) < f32(1.0))
                & (jnp.abs(xb[2:3]) < f32(1.0)))
        h1 = jnp.maximum(dg1(Wd1[...], f) + bd1[...], f32(0.0))
        h = dg0(Wd2[...], h1) + bd2[...]
        pe = [db]
        for j in range(4):
            s = f32(2.0 ** j)
            pe.append(jnp.sin(s * db))
            pe.append(jnp.cos(s * db))
        xi = jnp.concatenate(pe, axis=0)
        c1 = jnp.maximum(dg0(Wc1h[...], h) + dg0(Wc1x[...], xi) + bc1[...], f32(0.0))
        c2 = jnp.maximum(dg0(Wc2[...], c1) + bc2[...], f32(0.0))
        zc = dg0(Wc3[...], c2) + bc3[...]
        color = f32(1.0) / (f32(1.0) + jnp.exp(-zc))
        s1 = jnp.maximum(dg0(Ws1[...], h) + bs1[...], f32(0.0))
        s2 = jnp.maximum(dg0(Ws2[...], s1) + bs2[...], f32(0.0))
        zs = dg0(Ws3[...], s2) + bs3[...]
        zs = zs - jnp.max(zs, axis=0, keepdims=True)
        ez = jnp.exp(zs)
        seg = ez / jnp.sum(ez, axis=0, keepdims=True)
        color_r[...] = jnp.where(mask, color, f32(0.0))
        seg_r[...] = jnp.where(mask, seg, f32(0.0))
        sigma_r[...] = jnp.exp(jnp.where(mask, h[0:1], f32(-100000.0)))

    zero = np.int32(0)

    def c_spec(rows):
        return pl.BlockSpec((rows, C, CP), lambda i: (zero, i, zero))

    def full_spec(arr):
        nd = arr.ndim
        return pl.BlockSpec(arr.shape, lambda i, _nd=nd: (zero,) * _nd)

    wlist = [w["Wd1"], w["bd1"], w["Wd2"], w["bd2"],
             w["Wc1h"], w["Wc1x"], w["bc1"], w["Wc2"], w["bc2"], w["Wc3"], w["bc3"],
             w["Ws1"], w["bs1"], w["Ws2"], w["bs2"], w["Ws3"], w["bs3"]]

    color, sigma, seg = pl.pallas_call(
        body,
        grid=grid,
        in_specs=[pl.BlockSpec((C, 32, CP), lambda i: (i, zero, zero)),
                  c_spec(3), c_spec(3)]
        + [full_spec(a) for a in wlist],
        out_specs=[c_spec(3), c_spec(1), c_spec(10)],
        out_shape=[
            jax.ShapeDtypeStruct((3, B // CP, CP), f32),
            jax.ShapeDtypeStruct((1, B // CP, CP), f32),
            jax.ShapeDtypeStruct((10, B // CP, CP), f32),
        ],
        compiler_params=pltpu.CompilerParams(
            dimension_semantics=("arbitrary",)),
    )(raw, xT3, dT3, *wlist)
    return color, sigma, seg


def kernel(x, d, tables, params):
    x = x.astype(f32)
    d = d.astype(f32)
    xs = (x * f32(0.5) + f32(0.5)).astype(f32)
    xT = xs.T
    tnative = tables.astype(f32).reshape(NLEV, T // 128, 128, 2).transpose(0, 1, 3, 2)
    tpack = _tc_pack(tnative).reshape(NLEV * T)
    raw = _sc_features(xT, tpack).reshape(B // CP, 2 * NLEV, CP)
    xT3 = x.T.reshape(3, B // CP, CP)
    dT3 = d.T.reshape(3, B // CP, CP)
    w = {
        "Wd1": params["Wd1"], "bd1": params["bd1"].reshape(-1, 1, 1),
        "Wd2": params["Wd2"], "bd2": params["bd2"].reshape(-1, 1, 1),
        "Wc1h": params["Wc1"][:16], "Wc1x": params["Wc1"][16:],
        "bc1": params["bc1"].reshape(-1, 1, 1),
        "Wc2": params["Wc2"], "bc2": params["bc2"].reshape(-1, 1, 1),
        "Wc3": params["Wc3"], "bc3": params["bc3"].reshape(-1, 1, 1),
        "Ws1": params["Ws1"], "bs1": params["bs1"].reshape(-1, 1, 1),
        "Ws2": params["Ws2"], "bs2": params["bs2"].reshape(-1, 1, 1),
        "Ws3": params["Ws3"], "bs3": params["bs3"].reshape(-1, 1, 1),
    }
    w = {k2: v.astype(f32) for k2, v in w.items()}
    colorT, sigmaT, segT = _tc_heads(raw, xT3, dT3, w)
    color = colorT.reshape(3, B).T
    sigma = sigmaT.reshape(B)
    seg = segT.reshape(10, B).T
    return color, sigma, seg


# speedup vs baseline: 1.0107x; 1.0107x over previous
import functools

import jax
import jax.numpy as jnp
import numpy as np
from jax import lax
from jax.experimental import pallas as pl
from jax.experimental.pallas import tpu as pltpu
from jax.experimental.pallas import tpu_sc as plsc

B = 65536
T = 524288
NL = [16, 22, 30, 42, 58, 80, 111, 153, 212, 294, 406, 561, 776, 1072, 1482, 2048]
NLEV = len(NL)
K2 = np.int32(np.int64(2654435761) - (1 << 32))
K3 = np.int32(805459861)
MASK19 = np.int32(T - 1)

NC, NS, LANES = 2, 16, 16
NW = NC * NS
PTS = B // NW
CP = 128
GP = 64
NCHUNK = PTS // GP
GROUPS = GP // LANES
NIDX = NLEV * 8

f32 = jnp.float32
i32 = jnp.int32


def _tc_pack(tables):
    NB = 2048
    zero = np.int32(0)

    def body(in_r, out_r):
        blk = in_r[...]
        a = blk[0, :, 0, :]
        b = blk[0, :, 1, :]
        a32 = lax.bitcast_convert_type(a.astype(jnp.bfloat16), jnp.uint16).astype(jnp.uint32)
        b32 = lax.bitcast_convert_type(b.astype(jnp.bfloat16), jnp.uint16).astype(jnp.uint32)
        word = (b32 << jnp.uint32(16)) | a32
        out_r[...] = lax.bitcast_convert_type(word, i32)[None]

    return pl.pallas_call(
        body,
        grid=(NLEV, (T // 128) // NB),
        in_specs=[pl.BlockSpec((1, NB, 2, 128), lambda i, j: (i, j, zero, zero))],
        out_specs=pl.BlockSpec((1, NB, 128), lambda i, j: (i, j, zero)),
        out_shape=jax.ShapeDtypeStruct((NLEV, T // 128, 128), i32),
        compiler_params=pltpu.CompilerParams(
            dimension_semantics=("arbitrary", "arbitrary")),
    )(tables)


def _sc_features(xT, tpack):
    mesh = plsc.VectorSubcoreMesh(core_axis_name="c", subcore_axis_name="s")

    @functools.partial(
        pl.kernel,
        mesh=mesh,
        out_type=jax.ShapeDtypeStruct((B * 2 * NLEV,), f32),
        scratch_types=[
            pltpu.VMEM((3, PTS), f32),
            pltpu.VMEM((NIDX * GP,), i32),
            pltpu.VMEM((NIDX * GP,), i32),
            pltpu.VMEM((NIDX * GP,), i32),
            pltpu.VMEM((NIDX * GP,), i32),
            pltpu.VMEM((CP * 2 * NLEV,), f32),
            pltpu.SemaphoreType.DMA,
            pltpu.SemaphoreType.DMA,
        ],
    )
    def k(xT_hbm, tab_hbm, out_hbm, xyz_v, idxA, idxB, rowsA, rowsB, feat_v, semA, semB):
        wid = lax.axis_index("s") * NC + lax.axis_index("c")
        base = wid * PTS
        pltpu.sync_copy(xT_hbm.at[:, pl.ds(base, PTS)], xyz_v)

        def pass1(c, idx_v):
            cbase = c * GP

            def grp(g, carry1):
                p0 = g * LANES
                xs = xyz_v[0, pl.ds(cbase + p0, LANES)]
                ys = xyz_v[1, pl.ds(cbase + p0, LANES)]
                zs = xyz_v[2, pl.ds(cbase + p0, LANES)]
                for l in range(NLEV):
                    n = f32(NL[l])
                    ix = (xs * n).astype(i32)
                    iy = (ys * n).astype(i32)
                    iz = (zs * n).astype(i32)
                    hx0 = ix
                    hx1 = ix + 1
                    hy0 = iy * K2
                    hy1 = hy0 + K2
                    hz0 = iz * K3
                    hz1 = hz0 + K3
                    lbase = np.int32(l * T)
                    for v in range(8):
                        hx = hx1 if (v & 1) else hx0
                        hy = hy1 if (v >> 1) & 1 else hy0
                        hz = hz1 if (v >> 2) & 1 else hz0
                        e0 = ((hx ^ hy ^ hz) & MASK19) + lbase
                        s0 = (l * 8 + v) * GP
                        idx_v[pl.ds(s0 + p0, LANES)] = e0
                return carry1

            lax.fori_loop(i32(0), i32(GROUPS), grp, i32(0))

        def fire(idx_v, rows_v, sem):
            pltpu.async_copy(tab_hbm.at[idx_v], rows_v, sem)

        def wait(idx_v, rows_v, sem):
            pltpu.make_async_copy(tab_hbm.at[idx_v], rows_v, sem).wait()

        def pass2(c, rows_v, half):
            cbase = c * GP

            def grp(g, carry2):
                p0 = g * LANES
                xs = xyz_v[0, pl.ds(cbase + p0, LANES)]
                ys = xyz_v[1, pl.ds(cbase + p0, LANES)]
                zs = xyz_v[2, pl.ds(cbase + p0, LANES)]
                for l in range(NLEV):
                    n = f32(NL[l])
                    xn = xs * n
                    yn = ys * n
                    zn = zs * n
                    fx = xn - xn.astype(i32).astype(f32)
                    fy = yn - yn.astype(i32).astype(f32)
                    fz = zn - zn.astype(i32).astype(f32)
                    gx = f32(1.0) - fx
                    gy = f32(1.0) - fy
                    gz = f32(1.0) - fz
                    wxy = (gx * gy, fx * gy, gx * fy, fx * fy)
                    acc0 = jnp.zeros((LANES,), f32)
                    acc1 = jnp.zeros((LANES,), f32)
                    for v in range(8):
                        w = wxy[v & 3] * (fz if (v >> 2) & 1 else gz)
                        s0 = (l * 8 + v) * GP
                        r = rows_v[pl.ds(s0 + p0, LANES)]
                        f0 = lax.bitcast_convert_type(r << np.int32(16), f32)
                        f1 = lax.bitcast_convert_type(r & np.int32(-65536), f32)
                        acc0 = acc0 + f0 * w
                        acc1 = acc1 + f1 * w
                    feat_v[pl.ds((2 * l) * CP + half * GP + p0, LANES)] = acc0
                    feat_v[pl.ds((2 * l + 1) * CP + half * GP + p0, LANES)] = acc1
                return carry2

            lax.fori_loop(i32(0), i32(GROUPS), grp, i32(0))

        def flush(i):
            off = (wid * (PTS // CP) + i) * (CP * 2 * NLEV)
            pltpu.sync_copy(feat_v, out_hbm.at[pl.ds(off, CP * 2 * NLEV)])

        pass1(i32(0), idxA)
        fire(idxA, rowsA, semA)
        pass1(i32(1), idxB)

        def pair(i, carry):
            c = i * 2
            wait(idxA, rowsA, semA)
            fire(idxB, rowsB, semB)
            pass2(c, rowsA, 0)
            pass1(c + 2, idxA)
            wait(idxB, rowsB, semB)
            fire(idxA, rowsA, semA)
            pass2(c + 1, rowsB, 1)
            pass1(c + 3, idxB)
            flush(i)
            return carry

        lax.fori_loop(i32(0), i32(NCHUNK // 2 - 1), pair, i32(0))
        wait(idxA, rowsA, semA)
        fire(idxB, rowsB, semB)
        pass2(i32(NCHUNK - 2), rowsA, 0)
        wait(idxB, rowsB, semB)
        pass2(i32(NCHUNK - 1), rowsB, 1)
        flush(i32(NCHUNK // 2 - 1))

    return k(xT, tpack)


def _tc_heads(raw, xT3, dT3, w):
    TB = 4096
    C = TB // CP
    grid = (B // TB,)

    def dg1(wm, a):
        return lax.dot_general(wm, a, (((0,), (1,)), ((), ())),
                               preferred_element_type=f32)

    def dg0(wm, a):
        return lax.dot_general(wm, a, (((0,), (0,)), ((), ())),
                               preferred_element_type=f32)

    def body(raw_r, x_r, d_r,
             Wd1, bd1, Wd2, bd2, Wc1h, Wc1x, bc1, Wc2, bc2, Wc3, bc3,
             Ws1, bs1, Ws2, bs2, Ws3, bs3,
             color_r, sigma_r, seg_r):
        f = raw_r[...]
        xb = x_r[...]
        db = d_r[...]
        mask = ((jnp.abs(xb[0:1]) < f32(1.0))
                & (jnp.abs(xb[1:2]) < f32(1.0))
                & (jnp.abs(xb[2:3]) < f32(1.0)))
        h1 = jnp.maximum(dg1(Wd1[...], f) + bd1[...], f32(0.0))
        h = dg0(Wd2[...], h1) + bd2[...]
        pe = [db]
        for j in range(4):
            s = f32(2.0 ** j)
            pe.append(jnp.sin(s * db))
            pe.append(jnp.cos(s * db))
        xi = jnp.concatenate(pe, axis=0)
        c1 = jnp.maximum(dg0(Wc1h[...], h) + dg0(Wc1x[...], xi) + bc1[...], f32(0.0))
        c2 = jnp.maximum(dg0(Wc2[...], c1) + bc2[...], f32(0.0))
        zc = dg0(Wc3[...], c2) + bc3[...]
        color = f32(1.0) / (f32(1.0) + jnp.exp(-zc))
        s1 = jnp.maximum(dg0(Ws1[...], h) + bs1[...], f32(0.0))
        s2 = jnp.maximum(dg0(Ws2[...], s1) + bs2[...], f32(0.0))
        zs = dg0(Ws3[...], s2) + bs3[...]
        zs = zs - jnp.max(zs, axis=0, keepdims=True)
        ez = jnp.exp(zs)
        seg = ez / jnp.sum(ez, axis=0, keepdims=True)
        color_r[...] = jnp.where(mask, color, f32(0.0))
        seg_r[...] = jnp.where(mask, seg, f32(0.0))
        sigma_r[...] = jnp.exp(jnp.where(mask, h[0:1], f32(-100000.0)))

    zero = np.int32(0)

    def c_spec(rows):
        return pl.BlockSpec((rows, C, CP), lambda i: (zero, i, zero))

    def full_spec(arr):
        nd = arr.ndim
        return pl.BlockSpec(arr.shape, lambda i, _nd=nd: (zero,) * _nd)

    wlist = [w["Wd1"], w["bd1"], w["Wd2"], w["bd2"],
             w["Wc1h"], w["Wc1x"], w["bc1"], w["Wc2"], w["bc2"], w["Wc3"], w["bc3"],
             w["Ws1"], w["bs1"], w["Ws2"], w["bs2"], w["Ws3"], w["bs3"]]

    color, sigma, seg = pl.pallas_call(
        body,
        grid=grid,
        in_specs=[pl.BlockSpec((C, 32, CP), lambda i: (i, zero, zero)),
                  c_spec(3), c_spec(3)]
        + [full_spec(a) for a in wlist],
        out_specs=[c_spec(3), c_spec(1), c_spec(10)],
        out_shape=[
            jax.ShapeDtypeStruct((3, B // CP, CP), f32),
            jax.ShapeDtypeStruct((1, B // CP, CP), f32),
            jax.ShapeDtypeStruct((10, B // CP, CP), f32),
        ],
        compiler_params=pltpu.CompilerParams(
            dimension_semantics=("arbitrary",)),
    )(raw, xT3, dT3, *wlist)
    return color, sigma, seg


def kernel(x, d, tables, params):
    x = x.astype(f32)
    d = d.astype(f32)
    xs = (x * f32(0.5) + f32(0.5)).astype(f32)
    xT = xs.T
    tnative = tables.astype(f32).reshape(NLEV, T // 128, 128, 2).transpose(0, 1, 3, 2)
    tpack = _tc_pack(tnative).reshape(NLEV * T)
    raw = _sc_features(xT, tpack).reshape(B // CP, 2 * NLEV, CP)
    xT3 = x.T.reshape(3, B // CP, CP)
    dT3 = d.T.reshape(3, B // CP, CP)
    w = {
        "Wd1": params["Wd1"], "bd1": params["bd1"].reshape(-1, 1, 1),
        "Wd2": params["Wd2"], "bd2": params["bd2"].reshape(-1, 1, 1),
        "Wc1h": params["Wc1"][:16], "Wc1x": params["Wc1"][16:],
        "bc1": params["bc1"].reshape(-1, 1, 1),
        "Wc2": params["Wc2"], "bc2": params["bc2"].reshape(-1, 1, 1),
        "Wc3": params["Wc3"], "bc3": params["bc3"].reshape(-1, 1, 1),
        "Ws1": params["Ws1"], "bs1": params["bs1"].reshape(-1, 1, 1),
        "Ws2": params["Ws2"], "bs2": params["bs2"].reshape(-1, 1, 1),
        "Ws3": params["Ws3"], "bs3": params["bs3"].reshape(-1, 1, 1),
    }
    w = {k2: v.astype(f32) for k2, v in w.items()}
    colorT, sigmaT, segT = _tc_heads(raw, xT3, dT3, w)
    color = colorT.reshape(3, B).T
    sigma = sigmaT.reshape(B)
    seg = segT.reshape(10, B).T
    return color, sigma, seg
